# baseline (speedup 1.0000x reference)
import functools

import jax
import jax.numpy as jnp
from jax import lax
from jax.experimental import pallas as pl
from jax.experimental.pallas import tpu as pltpu
from jax.experimental.pallas import tpu_sc as plsc

B = 4096
F = 26
L = 20
D = 64
FL = F * L
OUT_D = F * D
LANES = 16
NC, NS = 2, 16
NW = NC * NS
BPW = B // NW
CHUNK = 104
NCHUNK = FL // CHUNK
HALVES = D // 32
GRP = 32
NGRP = BPW // GRP
NORM_BLK = 256

VOCAB = 100000
CROWS = 125
RPW = VOCAB // NW
NCAST = RPW // CROWS
CWORDS = CROWS * D


def _cast_body(tab_hbm, out_hbm, ins, outs, isems, osems):
    wid = lax.axis_index("s") * NC + lax.axis_index("c")
    r0 = wid * RPW

    def fire_load(i, buf):
        pltpu.async_copy(
            tab_hbm.at[pl.ds((r0 + i * CROWS) * D, CWORDS)], ins[buf], isems[buf]
        )

    def chunk(i, buf):
        pltpu.make_async_copy(
            tab_hbm.at[pl.ds(0, CWORDS)], ins[buf], isems[buf]
        ).wait()

        @pl.when(i >= 2)
        def _():
            pltpu.make_async_copy(
                outs[buf], out_hbm.at[pl.ds(0, CROWS)], osems[buf]
            ).wait()

        def row_body(r25, carry):
            for dr in range(25):
                r = r25 * 25 + dr
                o = r * D
                for c in range(HALVES):
                    a = ins[buf][pl.ds(o + c * 32, LANES)]
                    b = ins[buf][pl.ds(o + c * 32 + LANES, LANES)]
                    outs[buf][r, pl.ds(c * 32, 32)] = plsc.pack(
                        a, b, format=plsc.PackFormat.INTERLEAVED
                    )
            return carry

        lax.fori_loop(0, CROWS // 25, row_body, 0)

        @pl.when(i + 2 < NCAST)
        def _():
            fire_load(i + 2, buf)

        pltpu.async_copy(
            outs[buf], out_hbm.at[pl.ds(r0 + i * CROWS, CROWS)], osems[buf]
        )

    fire_load(0, 0)
    fire_load(1, 1)

    def pair_body(p, carry):
        chunk(2 * p, 0)
        chunk(2 * p + 1, 1)
        return carry

    lax.fori_loop(0, NCAST // 2, pair_body, 0)
    chunk(NCAST - 1, 0)
    for buf in range(2):
        pltpu.make_async_copy(
            outs[buf], out_hbm.at[pl.ds(0, CROWS)], osems[buf]
        ).wait()


def _xlane_sum(x):
    dnums = lax.GatherDimensionNumbers(
        offset_dims=(), collapsed_slice_dims=(0,), start_index_map=(0,)
    )
    for k in (1, 2, 4, 8):
        idx = lax.iota(jnp.int32, LANES) ^ k
        shuf = lax.gather(
            x,
            idx[:, None],
            dimension_numbers=dnums,
            slice_sizes=(1,),
            mode=lax.GatherScatterMode.PROMISE_IN_BOUNDS,
        )
        x = x + shuf
    return x


def _rsqrt_newton(v):
    i = plsc.bitcast(v, jnp.int32)
    seed = jnp.full((LANES,), 0x5F3759DF, dtype=jnp.int32) - lax.shift_right_logical(i, 1)
    y = plsc.bitcast(seed, jnp.float32)
    for _ in range(3):
        y = y * (1.5 - 0.5 * v * y * y)
    return y


def _sc_body(x_hbm, table_hbm, out_hbm, idx_v, rows, outs, gsems, osems):
    wid = lax.axis_index("s") * NC + lax.axis_index("c")
    base = wid * BPW

    def fire_gathers(j, rbuf):
        for k in range(NCHUNK):
            pltpu.async_copy(
                table_hbm.at[idx_v.at[j, k]],
                rows[rbuf].at[pl.ds(k * CHUNK, CHUNK)],
                gsems[rbuf],
            )

    def drain_gathers(rbuf):
        pltpu.make_async_copy(
            table_hbm.at[pl.ds(0, FL)], rows[rbuf], gsems[rbuf]
        ).wait()

    def out_slice(row):
        return out_hbm.at[row // 8, :, row % 8, :]

    def drain_store(obuf, row):
        pltpu.make_async_copy(outs[obuf], out_slice(row), osems[obuf]).wait()

    def compute(rbuf, obuf):
        rows_ref = rows[rbuf]
        out_ref = outs[obuf]

        def field_body(f, carry):
            r0 = f * L
            acc = [
                [jnp.zeros((LANES,), jnp.float32) for _ in range(2)]
                for _ in range(HALVES)
            ]
            for l in range(0, L, 2):
                for c in range(HALVES):
                    s = (
                        rows_ref[r0 + l, pl.ds(c * 32, 32)]
                        + rows_ref[r0 + l + 1, pl.ds(c * 32, 32)]
                    )
                    lo, hi = plsc.unpack(
                        s,
                        format=plsc.PackFormat.INTERLEAVED,
                        preferred_element_type=jnp.float32,
                    )
                    acc[c][0] = acc[c][0] + lo
                    acc[c][1] = acc[c][1] + hi
            orow = f // 2
            ocol = pl.multiple_of((f % 2) * D, D)
            ss = carry
            for c in range(HALVES):
                for h in range(2):
                    out_ref[orow, pl.ds(ocol + c * 32 + h * LANES, LANES)] = acc[c][h]
                    ss = ss + acc[c][h] * acc[c][h]
            return ss

        ss = lax.fori_loop(0, F, field_body, jnp.zeros((LANES,), jnp.float32))
        v = jnp.maximum(_xlane_sum(ss), 1e-24)
        scale = jnp.minimum(_rsqrt_newton(v), 1e12)

        def scale_body(r, carry):
            for cc in range(8):
                o = cc * LANES
                out_ref[r, pl.ds(o, LANES)] = out_ref[r, pl.ds(o, LANES)] * scale
            return carry

        lax.fori_loop(0, 13, scale_body, 0)

    def load_idx_and_prime(g):
        pltpu.sync_copy(x_hbm.at[pl.ds(base + g * GRP, GRP)], idx_v)
        for j in range(4):
            fire_gathers(j, j)

    load_idx_and_prime(0)

    def group_body(g, carry):
        for j in range(GRP):
            bj = base + g * GRP + j
            rbuf = j % 4
            obuf = j % 2
            drain_gathers(rbuf)
            if j < 2:

                @pl.when(g > 0)
                def _():
                    drain_store(obuf, bj - 2)

            else:
                drain_store(obuf, bj - 2)
            compute(rbuf, obuf)
            pltpu.async_copy(outs[obuf], out_slice(bj), osems[obuf])
            if j + 4 < GRP:
                fire_gathers(j + 4, rbuf)

        @pl.when(g + 1 < NGRP)
        def _():
            load_idx_and_prime(g + 1)

        return carry

    lax.fori_loop(0, NGRP, group_body, 0)
    drain_store(0, base + BPW - 2)
    drain_store(1, base + BPW - 1)


@jax.jit
def _run(x2, table_flat):
    mesh = plsc.VectorSubcoreMesh(core_axis_name="c", subcore_axis_name="s")
    cparams = pltpu.CompilerParams(
        use_tc_tiling_on_sc=False, needs_layout_passes=False
    )
    tableb = functools.partial(
        pl.kernel,
        mesh=mesh,
        out_type=jax.ShapeDtypeStruct((VOCAB, D), jnp.bfloat16),
        scratch_types=[
            [pltpu.VMEM((CWORDS,), jnp.float32) for _ in range(2)],
            [pltpu.VMEM((CROWS, D), jnp.bfloat16) for _ in range(2)],
            [pltpu.SemaphoreType.DMA for _ in range(2)],
            [pltpu.SemaphoreType.DMA for _ in range(2)],
        ],
        compiler_params=cparams,
    )(_cast_body)(table_flat)
    return functools.partial(
        pl.kernel,
        mesh=mesh,
        out_type=jax.ShapeDtypeStruct((B // 8, 13, 8, 128), jnp.float32),
        scratch_types=[
            pltpu.VMEM((GRP, NCHUNK, CHUNK), jnp.int32),
            [pltpu.VMEM((FL, D), jnp.bfloat16) for _ in range(4)],
            [pltpu.VMEM((13, 128), jnp.float32) for _ in range(2)],
            [pltpu.SemaphoreType.DMA for _ in range(4)],
            [pltpu.SemaphoreType.DMA for _ in range(2)],
        ],
        compiler_params=cparams,
    )(_sc_body)(x2, tableb)


def kernel(x, table):
    out4d = _run(x.reshape(B, NCHUNK, CHUNK), table.reshape(-1))
    return out4d.transpose(0, 2, 1, 3).reshape(B, OUT_D)


# speedup vs baseline: 1.0151x; 1.0151x over previous
import functools

import jax
import jax.numpy as jnp
from jax import lax
from jax.experimental import pallas as pl
from jax.experimental.pallas import tpu as pltpu
from jax.experimental.pallas import tpu_sc as plsc

B = 4096
F = 26
L = 20
D = 64
FL = F * L
OUT_D = F * D
LANES = 16
NC, NS = 2, 16
NW = NC * NS
BPW = B // NW
CHUNK = 104
NCHUNK = FL // CHUNK
HALVES = D // 32
GRP = 32
NGRP = BPW // GRP
NORM_BLK = 256

VOCAB = 100000
CROWS = 125
RPW = VOCAB // NW
NCAST = RPW // CROWS
CWORDS = CROWS * D


def _cast_body(tab_hbm, out_hbm, ins, outs, isems, osems):
    wid = lax.axis_index("s") * NC + lax.axis_index("c")
    r0 = wid * RPW

    def fire_load(i, buf):
        pltpu.async_copy(
            tab_hbm.at[pl.ds((r0 + i * CROWS) * D, CWORDS)], ins[buf], isems[buf]
        )

    def chunk(i, buf):
        pltpu.make_async_copy(
            tab_hbm.at[pl.ds(0, CWORDS)], ins[buf], isems[buf]
        ).wait()

        @pl.when(i >= 2)
        def _():
            pltpu.make_async_copy(
                outs[buf], out_hbm.at[pl.ds(0, CROWS)], osems[buf]
            ).wait()

        def row_body(r25, carry):
            for dr in range(25):
                r = r25 * 25 + dr
                o = r * D
                for c in range(HALVES):
                    a = ins[buf][pl.ds(o + c * 32, LANES)]
                    b = ins[buf][pl.ds(o + c * 32 + LANES, LANES)]
                    outs[buf][r, pl.ds(c * 32, 32)] = plsc.pack(
                        a, b, format=plsc.PackFormat.INTERLEAVED
                    )
            return carry

        lax.fori_loop(0, CROWS // 25, row_body, 0)

        @pl.when(i + 2 < NCAST)
        def _():
            fire_load(i + 2, buf)

        pltpu.async_copy(
            outs[buf], out_hbm.at[pl.ds(r0 + i * CROWS, CROWS)], osems[buf]
        )

    fire_load(0, 0)
    fire_load(1, 1)

    def pair_body(p, carry):
        chunk(2 * p, 0)
        chunk(2 * p + 1, 1)
        return carry

    lax.fori_loop(0, NCAST // 2, pair_body, 0)
    chunk(NCAST - 1, 0)
    for buf in range(2):
        pltpu.make_async_copy(
            outs[buf], out_hbm.at[pl.ds(0, CROWS)], osems[buf]
        ).wait()


def _xlane_sum(x):
    dnums = lax.GatherDimensionNumbers(
        offset_dims=(), collapsed_slice_dims=(0,), start_index_map=(0,)
    )
    for k in (1, 2, 4, 8):
        idx = lax.iota(jnp.int32, LANES) ^ k
        shuf = lax.gather(
            x,
            idx[:, None],
            dimension_numbers=dnums,
            slice_sizes=(1,),
            mode=lax.GatherScatterMode.PROMISE_IN_BOUNDS,
        )
        x = x + shuf
    return x


def _rsqrt_newton(v):
    i = plsc.bitcast(v, jnp.int32)
    seed = jnp.full((LANES,), 0x5F3759DF, dtype=jnp.int32) - lax.shift_right_logical(i, 1)
    y = plsc.bitcast(seed, jnp.float32)
    for _ in range(3):
        y = y * (1.5 - 0.5 * v * y * y)
    return y


def _sc_body(x_hbm, table_hbm, out_hbm, idx_v, rows, outs, gsems, osems):
    wid = lax.axis_index("s") * NC + lax.axis_index("c")
    base = wid * BPW

    def fire_gathers(j, rbuf):
        for k in range(NCHUNK):
            pltpu.async_copy(
                table_hbm.at[idx_v.at[j, k]],
                rows[rbuf].at[pl.ds(k * CHUNK, CHUNK)],
                gsems[rbuf],
            )

    def drain_gathers(rbuf):
        pltpu.make_async_copy(
            table_hbm.at[pl.ds(0, FL)], rows[rbuf], gsems[rbuf]
        ).wait()

    def out_slice(row):
        return out_hbm.at[row // 8, :, row % 8, :]

    def drain_store(obuf, row):
        pltpu.make_async_copy(outs[obuf], out_slice(row), osems[obuf]).wait()

    def compute(rbuf, obuf):
        rows_ref = rows[rbuf]
        out_ref = outs[obuf]

        def field_body(f, carry):
            r0 = f * L
            acc = [
                [jnp.zeros((LANES,), jnp.float32) for _ in range(2)]
                for _ in range(HALVES)
            ]
            for l in range(0, L, 2):
                for c in range(HALVES):
                    s = (
                        rows_ref[r0 + l, pl.ds(c * 32, 32)]
                        + rows_ref[r0 + l + 1, pl.ds(c * 32, 32)]
                    )
                    lo, hi = plsc.unpack(
                        s,
                        format=plsc.PackFormat.INTERLEAVED,
                        preferred_element_type=jnp.float32,
                    )
                    acc[c][0] = acc[c][0] + lo
                    acc[c][1] = acc[c][1] + hi
            orow = f // 2
            ocol = pl.multiple_of((f % 2) * D, D)
            ss = carry
            for c in range(HALVES):
                for h in range(2):
                    out_ref[orow, pl.ds(ocol + c * 32 + h * LANES, LANES)] = acc[c][h]
                    ss = ss + acc[c][h] * acc[c][h]
            return ss

        ss = lax.fori_loop(0, F, field_body, jnp.zeros((LANES,), jnp.float32))
        v = jnp.maximum(_xlane_sum(ss), 1e-24)
        scale = jnp.minimum(_rsqrt_newton(v), 1e12)

        def scale_body(r, carry):
            for cc in range(8):
                o = cc * LANES
                out_ref[r, pl.ds(o, LANES)] = out_ref[r, pl.ds(o, LANES)] * scale
            return carry

        lax.fori_loop(0, 13, scale_body, 0)

    def load_idx_and_prime(g):
        pltpu.sync_copy(x_hbm.at[pl.ds(base + g * GRP, GRP)], idx_v)
        for j in range(3):
            fire_gathers(j, j)

    load_idx_and_prime(0)

    def group_body(g, carry):
        for j in range(GRP):
            bj = base + g * GRP + j
            rbuf = j % 3
            obuf = j % 2
            drain_gathers(rbuf)
            if j < 2:

                @pl.when(g > 0)
                def _():
                    drain_store(obuf, bj - 2)

            else:
                drain_store(obuf, bj - 2)
            compute(rbuf, obuf)
            pltpu.async_copy(outs[obuf], out_slice(bj), osems[obuf])
            if j + 3 < GRP:
                fire_gathers(j + 3, rbuf)

        @pl.when(g + 1 < NGRP)
        def _():
            load_idx_and_prime(g + 1)

        return carry

    lax.fori_loop(0, NGRP, group_body, 0)
    drain_store(0, base + BPW - 2)
    drain_store(1, base + BPW - 1)


@jax.jit
def _run(x2, table_flat):
    mesh = plsc.VectorSubcoreMesh(core_axis_name="c", subcore_axis_name="s")
    cparams = pltpu.CompilerParams(
        use_tc_tiling_on_sc=False, needs_layout_passes=False
    )
    tableb = functools.partial(
        pl.kernel,
        mesh=mesh,
        out_type=jax.ShapeDtypeStruct((VOCAB, D), jnp.bfloat16),
        scratch_types=[
            [pltpu.VMEM((CWORDS,), jnp.float32) for _ in range(2)],
            [pltpu.VMEM((CROWS, D), jnp.bfloat16) for _ in range(2)],
            [pltpu.SemaphoreType.DMA for _ in range(2)],
            [pltpu.SemaphoreType.DMA for _ in range(2)],
        ],
        compiler_params=cparams,
    )(_cast_body)(table_flat)
    return functools.partial(
        pl.kernel,
        mesh=mesh,
        out_type=jax.ShapeDtypeStruct((B // 8, 13, 8, 128), jnp.float32),
        scratch_types=[
            pltpu.VMEM((GRP, NCHUNK, CHUNK), jnp.int32),
            [pltpu.VMEM((FL, D), jnp.bfloat16) for _ in range(3)],
            [pltpu.VMEM((13, 128), jnp.float32) for _ in range(2)],
            [pltpu.SemaphoreType.DMA for _ in range(3)],
            [pltpu.SemaphoreType.DMA for _ in range(2)],
        ],
        compiler_params=cparams,
    )(_sc_body)(x2, tableb)


def kernel(x, table):
    out4d = _run(x.reshape(B, NCHUNK, CHUNK), table.reshape(-1))
    return out4d.transpose(0, 2, 1, 3).reshape(B, OUT_D)
